# baseline (speedup 1.0000x reference)
import functools

import jax
import jax.numpy as jnp
from jax import lax
from jax.experimental import pallas as pl
from jax.experimental.pallas import tpu as pltpu
from jax.experimental.pallas import tpu_sc as plsc

_B = 4096
_S = 50
_N_DISC = 26
_N_CONT = 13
_VOCAB = 1000
_EMB = 8
_BS = _B * _S
_NX = _N_DISC + _N_CONT
_D_OUT = 64
_EPS = 1e-5

_NW = 32
_BPW = _B // _NW
_NB = 8
_TOK = _NB * _S
_NR = _TOK * 16
_NCHUNK = _BPW // _NB
_NG = _TOK // 16


def _sc_gather_fused(x3d, table):
    mesh = plsc.VectorSubcoreMesh(core_axis_name="c", subcore_axis_name="s")

    @functools.partial(
        pl.kernel,
        mesh=mesh,
        compiler_params=pltpu.CompilerParams(
            use_tc_tiling_on_sc=False, needs_layout_passes=False),
        out_type=(
            jax.ShapeDtypeStruct((_BS * 16, _EMB), jnp.float32),
            jax.ShapeDtypeStruct((_BS * 16, _EMB), jnp.float32),
        ),
        scratch_types=[
            pltpu.VMEM((_NB, _S, _NX), jnp.float32),
            pltpu.VMEM((_NR,), jnp.int32),
            pltpu.VMEM((_NR,), jnp.int32),
            pltpu.VMEM((_NR, _EMB), jnp.float32),
            pltpu.SemaphoreType.DMA,
        ],
    )
    def gather_kernel(x_hbm, table_hbm, ga_hbm, gb_hbm,
                      xbuf, idxa, idxb, rows, sem):
        wid = lax.axis_index("s") * 2 + lax.axis_index("c")
        batch_base = wid * _BPW
        lanes = lax.iota(jnp.int32, 16)
        zeros16 = lanes * 0

        def zfill(k, carry2):
            idxa[pl.ds(k * 16, 16)] = zeros16
            idxb[pl.ds(k * 16, 16)] = zeros16
            return carry2

        lax.fori_loop(0, _NR // 16, zfill, 0)

        def chunk_body(c, carry):
            b0 = batch_base + c * _NB
            tok0 = b0 * _S
            pltpu.sync_copy(x_hbm.at[pl.ds(b0, _NB)], xbuf)

            def group_body(g, carry2):
                tvec = g * 16 + lanes
                bv = tvec // _S
                sv = tvec % _S
                t16 = tvec * 16
                for f in range(16):
                    codes = plsc.load_gather(
                        xbuf, [bv, sv, jnp.full((16,), f, jnp.int32)])
                    plsc.store_scatter(
                        idxa, [t16 + f],
                        codes.astype(jnp.int32) + f * _VOCAB)
                for f in range(16, 26):
                    codes = plsc.load_gather(
                        xbuf, [bv, sv, jnp.full((16,), f, jnp.int32)])
                    plsc.store_scatter(
                        idxb, [t16 + (f - 16)],
                        codes.astype(jnp.int32) + f * _VOCAB)
                for m in range(10, 16):
                    plsc.store_scatter(idxb, [t16 + m], zeros16)
                return carry2


            pltpu.async_copy(table_hbm.at[idxa], rows, sem).wait()
            pltpu.sync_copy(rows, ga_hbm.at[pl.ds(tok0 * 16, _NR)])

            pltpu.async_copy(table_hbm.at[idxb], rows, sem).wait()

            def cont_body(g, carry2):
                tvec = g * 16 + lanes
                bv = tvec // _S
                sv = tvec % _S
                t16 = tvec * 16
                for c in range(_N_CONT):
                    cv = plsc.load_gather(
                        xbuf, [bv, sv,
                               jnp.full((16,), _N_DISC + c, jnp.int32)])
                    plsc.store_scatter(
                        rows,
                        [t16 + (10 + c // _EMB),
                         jnp.full((16,), c % _EMB, jnp.int32)], cv)
                return carry2

            pltpu.sync_copy(rows, gb_hbm.at[pl.ds(tok0 * 16, _NR)])
            return carry

        lax.fori_loop(0, _NCHUNK, chunk_body, 0)

    return gather_kernel(x3d, table)


_BB = 16


def _dense_body(ga_ref, gb_ref, wa_ref, wb_ref, b_ref, so_ref, to_ref,
                out_ref):
    wa = wa_ref[...]
    wb = wb_ref[...]
    bias = b_ref[...]
    so = so_ref[...]
    to = to_ref[...]
    for b in range(_BB):
        ga = ga_ref[b * _S:(b + 1) * _S, :]
        gb = gb_ref[b * _S:(b + 1) * _S, :]
        z = jnp.dot(ga, wa, preferred_element_type=jnp.float32)
        z = z + jnp.dot(gb, wb, preferred_element_type=jnp.float32)
        z = z + bias
        out_ref[b] = jnp.maximum(z, 0.0) * so + to


def _dense_call(ga2, gb2, wa, wb, b2, so, to):
    return pl.pallas_call(
        _dense_body,
        grid=(_B // _BB,),
        in_specs=[
            pl.BlockSpec((_BB * _S, 128), lambda i: (i, 0)),
            pl.BlockSpec((_BB * _S, 128), lambda i: (i, 0)),
            pl.BlockSpec((128, _D_OUT), lambda i: (0, 0)),
            pl.BlockSpec((128, _D_OUT), lambda i: (0, 0)),
            pl.BlockSpec((1, _D_OUT), lambda i: (0, 0)),
            pl.BlockSpec((1, _D_OUT), lambda i: (0, 0)),
            pl.BlockSpec((1, _D_OUT), lambda i: (0, 0)),
        ],
        out_specs=pl.BlockSpec((_BB, _S, _D_OUT), lambda i: (i, 0, 0)),
        out_shape=jax.ShapeDtypeStruct((_B, _S, _D_OUT), jnp.float32),
    )(ga2, gb2, wa, wb, b2, so, to)


def kernel(x, emb_tables, cbn_w, cbn_b, cbn_rm, cbn_rv, lin_W, lin_b,
           obn_w, obn_b, obn_rm, obn_rv):
    table = emb_tables.reshape(_N_DISC * _VOCAB, _EMB)

    s_c = cbn_w / jnp.sqrt(cbn_rv + _EPS)
    t_c = cbn_b - cbn_rm * s_c
    wc = lin_W[:, :_N_CONT]
    wc2 = (wc * s_c[None, :]).T
    b2 = lin_b + t_c @ wc.T

    wa = lin_W[:, _N_CONT:_N_CONT + 128].T
    wb = jnp.concatenate([
        lin_W[:, _N_CONT + 128:].T,
        wc2,
        jnp.zeros((35, _D_OUT), jnp.float32),
    ], axis=0)

    s_o = obn_w / jnp.sqrt(obn_rv + _EPS)
    t_o = obn_b - obn_rm * s_o

    ga, gb = _sc_gather_fused(x, table)
    ga2 = ga.reshape(_BS, 128)
    gb2 = gb.reshape(_BS, 128)

    return _dense_call(ga2, gb2, wa, wb,
                       b2.reshape(1, _D_OUT),
                       s_o.reshape(1, _D_OUT),
                       t_o.reshape(1, _D_OUT))


# speedup vs baseline: 29.4245x; 29.4245x over previous
import functools

import jax
import jax.numpy as jnp
from jax import lax
from jax.experimental import pallas as pl
from jax.experimental.pallas import tpu as pltpu
from jax.experimental.pallas import tpu_sc as plsc

_B = 4096
_S = 50
_N_DISC = 26
_N_CONT = 13
_VOCAB = 1000
_EMB = 8
_BS = _B * _S
_NX = _N_DISC + _N_CONT
_D_OUT = 64
_EPS = 1e-5

_NW = 32
_BPW = _B // _NW
_NB = 8
_TOK = _NB * _S
_NR = _TOK * 16
_NCHUNK = _BPW // _NB
_NG = _TOK // 16


def _sc_gather_fused(x3d, table):
    mesh = plsc.VectorSubcoreMesh(core_axis_name="c", subcore_axis_name="s")

    @functools.partial(
        pl.kernel,
        mesh=mesh,
        compiler_params=pltpu.CompilerParams(
            use_tc_tiling_on_sc=False, needs_layout_passes=False),
        out_type=(
            jax.ShapeDtypeStruct((_BS * 16, _EMB), jnp.float32),
            jax.ShapeDtypeStruct((_BS * 16, _EMB), jnp.float32),
        ),
        scratch_types=[
            pltpu.VMEM((_NB, _S, _NX), jnp.float32),
            pltpu.VMEM((_NR,), jnp.int32),
            pltpu.VMEM((_NR,), jnp.int32),
            pltpu.VMEM((_NR, _EMB), jnp.float32),
            pltpu.SemaphoreType.DMA,
        ],
    )
    def gather_kernel(x_hbm, table_hbm, ga_hbm, gb_hbm,
                      xbuf, idxa, idxb, rows, sem):
        wid = lax.axis_index("s") * 2 + lax.axis_index("c")
        batch_base = wid * _BPW
        lanes = lax.iota(jnp.int32, 16)
        zeros16 = lanes * 0

        def chunk_body(c, carry):
            b0 = batch_base + c * _NB
            tok0 = b0 * _S
            pltpu.sync_copy(x_hbm.at[pl.ds(b0, _NB)], xbuf)

            def group_body(g, carry2):
                tvec = g * 16 + lanes
                bv = tvec // _S
                sv = tvec % _S
                t16 = tvec * 16
                for f in range(16):
                    codes = plsc.load_gather(
                        xbuf, [bv, sv, jnp.full((16,), f, jnp.int32)])
                    plsc.store_scatter(
                        idxa, [t16 + f],
                        codes.astype(jnp.int32) + f * _VOCAB)
                for f in range(16, 26):
                    codes = plsc.load_gather(
                        xbuf, [bv, sv, jnp.full((16,), f, jnp.int32)])
                    plsc.store_scatter(
                        idxb, [t16 + (f - 16)],
                        codes.astype(jnp.int32) + f * _VOCAB)
                for m in range(10, 16):
                    plsc.store_scatter(idxb, [t16 + m], t16 + m)
                return carry2

            lax.fori_loop(0, _NG, group_body, 0)

            pltpu.async_copy(table_hbm.at[idxa], rows, sem).wait()
            pltpu.sync_copy(rows, ga_hbm.at[pl.ds(tok0 * 16, _NR)])

            pltpu.async_copy(table_hbm.at[idxb], rows, sem).wait()

            def cont_body(g, carry2):
                tvec = g * 16 + lanes
                bv = tvec // _S
                sv = tvec % _S
                t16 = tvec * 16
                for c in range(_N_CONT):
                    cv = plsc.load_gather(
                        xbuf, [bv, sv,
                               jnp.full((16,), _N_DISC + c, jnp.int32)])
                    plsc.store_scatter(
                        rows,
                        [t16 + (10 + c // _EMB),
                         jnp.full((16,), c % _EMB, jnp.int32)], cv)
                return carry2

            lax.fori_loop(0, _NG, cont_body, 0)
            pltpu.sync_copy(rows, gb_hbm.at[pl.ds(tok0 * 16, _NR)])
            return carry

        lax.fori_loop(0, _NCHUNK, chunk_body, 0)

    return gather_kernel(x3d, table)


_BB = 16


def _dense_body(ga_ref, gb_ref, wa_ref, wb_ref, b_ref, so_ref, to_ref,
                out_ref):
    wa = wa_ref[...]
    wb = wb_ref[...]
    bias = b_ref[...]
    so = so_ref[...]
    to = to_ref[...]
    for b in range(_BB):
        ga = ga_ref[b * _S:(b + 1) * _S, :]
        gb = gb_ref[b * _S:(b + 1) * _S, :]
        z = jnp.dot(ga, wa, preferred_element_type=jnp.float32)
        z = z + jnp.dot(gb, wb, preferred_element_type=jnp.float32)
        z = z + bias
        out_ref[b] = jnp.maximum(z, 0.0) * so + to


def _dense_call(ga2, gb2, wa, wb, b2, so, to):
    return pl.pallas_call(
        _dense_body,
        grid=(_B // _BB,),
        in_specs=[
            pl.BlockSpec((_BB * _S, 128), lambda i: (i, 0)),
            pl.BlockSpec((_BB * _S, 128), lambda i: (i, 0)),
            pl.BlockSpec((128, _D_OUT), lambda i: (0, 0)),
            pl.BlockSpec((128, _D_OUT), lambda i: (0, 0)),
            pl.BlockSpec((1, _D_OUT), lambda i: (0, 0)),
            pl.BlockSpec((1, _D_OUT), lambda i: (0, 0)),
            pl.BlockSpec((1, _D_OUT), lambda i: (0, 0)),
        ],
        out_specs=pl.BlockSpec((_BB, _S, _D_OUT), lambda i: (i, 0, 0)),
        out_shape=jax.ShapeDtypeStruct((_B, _S, _D_OUT), jnp.float32),
    )(ga2, gb2, wa, wb, b2, so, to)


def kernel(x, emb_tables, cbn_w, cbn_b, cbn_rm, cbn_rv, lin_W, lin_b,
           obn_w, obn_b, obn_rm, obn_rv):
    table = emb_tables.reshape(_N_DISC * _VOCAB, _EMB)

    s_c = cbn_w / jnp.sqrt(cbn_rv + _EPS)
    t_c = cbn_b - cbn_rm * s_c
    wc = lin_W[:, :_N_CONT]
    wc2 = (wc * s_c[None, :]).T
    b2 = lin_b + t_c @ wc.T

    wa = lin_W[:, _N_CONT:_N_CONT + 128].T
    wb = jnp.concatenate([
        lin_W[:, _N_CONT + 128:].T,
        wc2,
        jnp.zeros((35, _D_OUT), jnp.float32),
    ], axis=0)

    s_o = obn_w / jnp.sqrt(obn_rv + _EPS)
    t_o = obn_b - obn_rm * s_o

    ga, gb = _sc_gather_fused(x, table)
    ga2 = ga.reshape(_BS, 128)
    gb2 = gb.reshape(_BS, 128)

    return _dense_call(ga2, gb2, wa, wb,
                       b2.reshape(1, _D_OUT),
                       s_o.reshape(1, _D_OUT),
                       t_o.reshape(1, _D_OUT))


# speedup vs baseline: 32.9913x; 1.1212x over previous
import functools

import jax
import jax.numpy as jnp
from jax import lax
from jax.experimental import pallas as pl
from jax.experimental.pallas import tpu as pltpu
from jax.experimental.pallas import tpu_sc as plsc

_B = 4096
_S = 50
_N_DISC = 26
_N_CONT = 13
_VOCAB = 1000
_EMB = 8
_BS = _B * _S
_NX = _N_DISC + _N_CONT
_D_OUT = 64
_EPS = 1e-5

_NW = 32
_BPW = _B // _NW
_NB = 8
_TOK = _NB * _S
_NR = _TOK * 16
_NCHUNK = _BPW // _NB
_NG = _TOK // 16


def _sc_gather_fused(x1d, table):
    mesh = plsc.VectorSubcoreMesh(core_axis_name="c", subcore_axis_name="s")

    @functools.partial(
        pl.kernel,
        mesh=mesh,
        compiler_params=pltpu.CompilerParams(
            use_tc_tiling_on_sc=False, needs_layout_passes=False),
        out_type=(
            jax.ShapeDtypeStruct((_BS * 16, _EMB), jnp.float32),
            jax.ShapeDtypeStruct((_BS * 16, _EMB), jnp.float32),
        ),
        scratch_types=[
            pltpu.VMEM((_TOK * _NX,), jnp.float32),
            pltpu.VMEM((_NR,), jnp.int32),
            pltpu.VMEM((_NR,), jnp.int32),
            pltpu.VMEM((_NR, _EMB), jnp.float32),
            pltpu.VMEM((_NR, _EMB), jnp.float32),
            pltpu.SemaphoreType.DMA,
            pltpu.SemaphoreType.DMA,
        ],
    )
    def gather_kernel(x_hbm, table_hbm, ga_hbm, gb_hbm,
                      xbuf, idxa, idxb, rowsa, rowsb, sema, semb):
        wid = lax.axis_index("s") * 2 + lax.axis_index("c")
        tok_base = wid * _BPW * _S
        lanes = lax.iota(jnp.int32, 16)

        def chunk_body(c, carry):
            tok0 = tok_base + c * _TOK
            pltpu.sync_copy(x_hbm.at[pl.ds(tok0 * _NX, _TOK * _NX)], xbuf)

            def group_body(g, carry2):
                tvec = g * 16 + lanes
                t39 = tvec * _NX
                t16 = tvec * 16
                for f in range(16):
                    codes = plsc.load_gather(xbuf, [t39 + f])
                    plsc.store_scatter(
                        idxa, [t16 + f],
                        codes.astype(jnp.int32) + f * _VOCAB)
                for f in range(16, 26):
                    codes = plsc.load_gather(xbuf, [t39 + f])
                    plsc.store_scatter(
                        idxb, [t16 + (f - 16)],
                        codes.astype(jnp.int32) + f * _VOCAB)
                for m in range(10, 16):
                    plsc.store_scatter(idxb, [t16 + m], t16 + m)
                return carry2

            lax.fori_loop(0, _NG, group_body, 0)

            cpa = pltpu.async_copy(table_hbm.at[idxa], rowsa, sema)
            cpb = pltpu.async_copy(table_hbm.at[idxb], rowsb, semb)
            cpa.wait()
            pltpu.sync_copy(rowsa, ga_hbm.at[pl.ds(tok0 * 16, _NR)])
            cpb.wait()

            def cont_body(g, carry2):
                tvec = g * 16 + lanes
                t39 = tvec * _NX
                t16 = tvec * 16
                for c in range(_N_CONT):
                    cv = plsc.load_gather(xbuf, [t39 + (_N_DISC + c)])
                    plsc.store_scatter(
                        rowsb,
                        [t16 + (10 + c // _EMB),
                         jnp.full((16,), c % _EMB, jnp.int32)], cv)
                return carry2

            lax.fori_loop(0, _NG, cont_body, 0)
            pltpu.sync_copy(rowsb, gb_hbm.at[pl.ds(tok0 * 16, _NR)])
            return carry

        lax.fori_loop(0, _NCHUNK, chunk_body, 0)

    return gather_kernel(x1d, table)


_BB = 16


def _dense_body(ga_ref, gb_ref, wa_ref, wb_ref, b_ref, so_ref, to_ref,
                out_ref):
    wa = wa_ref[...]
    wb = wb_ref[...]
    bias = b_ref[...]
    so = so_ref[...]
    to = to_ref[...]
    for b in range(_BB):
        ga = ga_ref[b * _S:(b + 1) * _S, :]
        gb = gb_ref[b * _S:(b + 1) * _S, :]
        z = jnp.dot(ga, wa, preferred_element_type=jnp.float32)
        z = z + jnp.dot(gb, wb, preferred_element_type=jnp.float32)
        z = z + bias
        out_ref[b] = jnp.maximum(z, 0.0) * so + to


def _dense_call(ga2, gb2, wa, wb, b2, so, to):
    return pl.pallas_call(
        _dense_body,
        grid=(_B // _BB,),
        in_specs=[
            pl.BlockSpec((_BB * _S, 128), lambda i: (i, 0)),
            pl.BlockSpec((_BB * _S, 128), lambda i: (i, 0)),
            pl.BlockSpec((128, _D_OUT), lambda i: (0, 0)),
            pl.BlockSpec((128, _D_OUT), lambda i: (0, 0)),
            pl.BlockSpec((1, _D_OUT), lambda i: (0, 0)),
            pl.BlockSpec((1, _D_OUT), lambda i: (0, 0)),
            pl.BlockSpec((1, _D_OUT), lambda i: (0, 0)),
        ],
        out_specs=pl.BlockSpec((_BB, _S, _D_OUT), lambda i: (i, 0, 0)),
        out_shape=jax.ShapeDtypeStruct((_B, _S, _D_OUT), jnp.float32),
    )(ga2, gb2, wa, wb, b2, so, to)


def kernel(x, emb_tables, cbn_w, cbn_b, cbn_rm, cbn_rv, lin_W, lin_b,
           obn_w, obn_b, obn_rm, obn_rv):
    table = emb_tables.reshape(_N_DISC * _VOCAB, _EMB)

    s_c = cbn_w / jnp.sqrt(cbn_rv + _EPS)
    t_c = cbn_b - cbn_rm * s_c
    wc = lin_W[:, :_N_CONT]
    wc2 = (wc * s_c[None, :]).T
    b2 = lin_b + t_c @ wc.T

    wa = lin_W[:, _N_CONT:_N_CONT + 128].T
    wb = jnp.concatenate([
        lin_W[:, _N_CONT + 128:].T,
        wc2,
        jnp.zeros((35, _D_OUT), jnp.float32),
    ], axis=0)

    s_o = obn_w / jnp.sqrt(obn_rv + _EPS)
    t_o = obn_b - obn_rm * s_o

    ga, gb = _sc_gather_fused(x.reshape(_BS * _NX), table)
    ga2 = ga.reshape(_BS, 128)
    gb2 = gb.reshape(_BS, 128)

    return _dense_call(ga2, gb2, wa, wb,
                       b2.reshape(1, _D_OUT),
                       s_o.reshape(1, _D_OUT),
                       t_o.reshape(1, _D_OUT))


# speedup vs baseline: 37.7539x; 1.1444x over previous
import functools

import jax
import jax.numpy as jnp
from jax import lax
from jax.experimental import pallas as pl
from jax.experimental.pallas import tpu as pltpu
from jax.experimental.pallas import tpu_sc as plsc

_B = 4096
_S = 50
_N_DISC = 26
_N_CONT = 13
_VOCAB = 1000
_EMB = 8
_BS = _B * _S
_NX = _N_DISC + _N_CONT
_D_OUT = 64
_EPS = 1e-5

_NW = 32
_BPW = _B // _NW
_NB = 8
_TOK = _NB * _S
_NR = _TOK * 16
_NCHUNK = _BPW // _NB
_NG = _TOK // 16


def _sc_gather_fused(xt, table):
    mesh = plsc.VectorSubcoreMesh(core_axis_name="c", subcore_axis_name="s")

    @functools.partial(
        pl.kernel,
        mesh=mesh,
        compiler_params=pltpu.CompilerParams(
            use_tc_tiling_on_sc=False, needs_layout_passes=False),
        out_type=(
            jax.ShapeDtypeStruct((_BS * 16, _EMB), jnp.float32),
            jax.ShapeDtypeStruct((_BS * 16, _EMB), jnp.float32),
        ),
        scratch_types=[
            pltpu.VMEM((_S, _NX, _NB), jnp.float32),
            pltpu.VMEM((_NR,), jnp.int32),
            pltpu.VMEM((_NR,), jnp.int32),
            pltpu.VMEM((_NR, _EMB), jnp.float32),
            pltpu.VMEM((_NR, _EMB), jnp.float32),
            pltpu.SemaphoreType.DMA,
            pltpu.SemaphoreType.DMA,
        ],
    )
    def gather_kernel(xt_hbm, table_hbm, ga_hbm, gb_hbm,
                      xbuf, idxa, idxb, rowsa, rowsb, sema, semb):
        wid = lax.axis_index("s") * 2 + lax.axis_index("c")
        b_base = wid * _BPW
        lanes = lax.iota(jnp.int32, 16)
        bpat = lanes % _NB
        hpat = lanes // _NB
        hoff = hpat * _VOCAB
        pos_pat = bpat * (_S * 16) + hpat

        def chunk_body(c, carry):
            b0 = b_base + c * _NB
            pltpu.sync_copy(xt_hbm.at[:, :, pl.ds(b0, _NB)], xbuf)

            def s_body(s, carry2):
                s16 = s * 16
                sv = jnp.full((16,), s, jnp.int32)
                for p in range(8):
                    f = 2 * p
                    codes = plsc.load_gather(xbuf, [sv, hpat + f, bpat])
                    plsc.store_scatter(
                        idxa, [pos_pat + (s16 + f)],
                        codes.astype(jnp.int32) + f * _VOCAB + hoff)
                for p in range(5):
                    f = 16 + 2 * p
                    codes = plsc.load_gather(xbuf, [sv, hpat + f, bpat])
                    plsc.store_scatter(
                        idxb, [pos_pat + (s16 + (f - 16))],
                        codes.astype(jnp.int32) + f * _VOCAB + hoff)
                for j in range(3):
                    posd = bpat * (_S * 16) + (s16 + 10 + 2 * j) + hpat
                    plsc.store_scatter(idxb, [posd], posd)
                return carry2

            lax.fori_loop(0, _S, s_body, 0)

            cpa = pltpu.async_copy(table_hbm.at[idxa], rowsa, sema)
            cpb = pltpu.async_copy(table_hbm.at[idxb], rowsb, semb)
            cpa.wait()
            pltpu.sync_copy(rowsa, ga_hbm.at[pl.ds(b0 * (_S * 16), _NR)])
            cpb.wait()

            def cont_body(s, carry2):
                s16 = s * 16
                sv = jnp.full((16,), s, jnp.int32)
                for p in range(7):
                    c0 = _N_DISC + 2 * p
                    fv = jnp.minimum(hpat + c0, _NX - 1)
                    cv = plsc.load_gather(xbuf, [sv, fv, bpat])
                    cidx = 2 * p + hpat
                    rowv = (bpat * (_S * 16) + (s16 + 10)
                            + cidx // _EMB)
                    plsc.store_scatter(rowsb, [rowv, cidx % _EMB], cv)
                return carry2

            lax.fori_loop(0, _S, cont_body, 0)
            pltpu.sync_copy(rowsb, gb_hbm.at[pl.ds(b0 * (_S * 16), _NR)])
            return carry

        lax.fori_loop(0, _NCHUNK, chunk_body, 0)

    return gather_kernel(xt, table)


_BB = 16


def _dense_body(ga_ref, gb_ref, wa_ref, wb_ref, b_ref, so_ref, to_ref,
                out_ref):
    wa = wa_ref[...]
    wb = wb_ref[...]
    bias = b_ref[...]
    so = so_ref[...]
    to = to_ref[...]
    for b in range(_BB):
        ga = ga_ref[b * _S:(b + 1) * _S, :]
        gb = gb_ref[b * _S:(b + 1) * _S, :]
        z = jnp.dot(ga, wa, preferred_element_type=jnp.float32)
        z = z + jnp.dot(gb, wb, preferred_element_type=jnp.float32)
        z = z + bias
        out_ref[b] = jnp.maximum(z, 0.0) * so + to


def _dense_call(ga2, gb2, wa, wb, b2, so, to):
    return pl.pallas_call(
        _dense_body,
        grid=(_B // _BB,),
        in_specs=[
            pl.BlockSpec((_BB * _S, 128), lambda i: (i, 0)),
            pl.BlockSpec((_BB * _S, 128), lambda i: (i, 0)),
            pl.BlockSpec((128, _D_OUT), lambda i: (0, 0)),
            pl.BlockSpec((128, _D_OUT), lambda i: (0, 0)),
            pl.BlockSpec((1, _D_OUT), lambda i: (0, 0)),
            pl.BlockSpec((1, _D_OUT), lambda i: (0, 0)),
            pl.BlockSpec((1, _D_OUT), lambda i: (0, 0)),
        ],
        out_specs=pl.BlockSpec((_BB, _S, _D_OUT), lambda i: (i, 0, 0)),
        out_shape=jax.ShapeDtypeStruct((_B, _S, _D_OUT), jnp.float32),
    )(ga2, gb2, wa, wb, b2, so, to)


def kernel(x, emb_tables, cbn_w, cbn_b, cbn_rm, cbn_rv, lin_W, lin_b,
           obn_w, obn_b, obn_rm, obn_rv):
    table = emb_tables.reshape(_N_DISC * _VOCAB, _EMB)

    s_c = cbn_w / jnp.sqrt(cbn_rv + _EPS)
    t_c = cbn_b - cbn_rm * s_c
    wc = lin_W[:, :_N_CONT]
    wc2 = (wc * s_c[None, :]).T
    b2 = lin_b + t_c @ wc.T

    wa = lin_W[:, _N_CONT:_N_CONT + 128].T
    wb = jnp.concatenate([
        lin_W[:, _N_CONT + 128:].T,
        wc2,
        jnp.zeros((35, _D_OUT), jnp.float32),
    ], axis=0)

    s_o = obn_w / jnp.sqrt(obn_rv + _EPS)
    t_o = obn_b - obn_rm * s_o

    ga, gb = _sc_gather_fused(jnp.transpose(x, (1, 2, 0)), table)
    ga2 = ga.reshape(_BS, 128)
    gb2 = gb.reshape(_BS, 128)

    return _dense_call(ga2, gb2, wa, wb,
                       b2.reshape(1, _D_OUT),
                       s_o.reshape(1, _D_OUT),
                       t_o.reshape(1, _D_OUT))
